# baseline (speedup 1.0000x reference)
import functools

import jax
import jax.numpy as jnp
from jax import lax
from jax.experimental import pallas as pl
from jax.experimental.pallas import tpu as pltpu
from jax.experimental.pallas import tpu_sc as plsc

B = 16384
K = 32
V = 1000000
PADK = 128
NC = 2
NS = 16
NW = NC * NS
BPW = B // NW
CHUNK = 128
NCHUNK = BPW // CHUNK

TBLK = 16384
QROWS = TBLK // 4
TGRID = (V + TBLK - 1) // TBLK
VPACK = TGRID * QROWS


def _transpose_body(ut_ref, it_ref, p_ref, uo_ref, io_ref):
    uo = io = None
    for q in range(4):
        pq = p_ref[q]
        du = lax.dot_general(
            ut_ref[:, pl.ds(q * QROWS, QROWS)].astype(jnp.bfloat16), pq,
            (((0,), (0,)), ((), ())), preferred_element_type=jnp.float32)
        di = lax.dot_general(
            it_ref[:, pl.ds(q * QROWS, QROWS)].astype(jnp.bfloat16), pq,
            (((0,), (0,)), ((), ())), preferred_element_type=jnp.float32)
        uo = du if uo is None else uo + du
        io = di if io is None else io + di
    uo_ref[...] = uo.astype(jnp.bfloat16)
    io_ref[...] = io.astype(jnp.bfloat16)


_transpose_call = pl.pallas_call(
    _transpose_body,
    grid=(TGRID,),
    in_specs=[
        pl.BlockSpec((K, TBLK), lambda i: (0, i)),
        pl.BlockSpec((K, TBLK), lambda i: (0, i)),
        pl.BlockSpec((4, K, PADK), lambda i: (0, 0, 0)),
    ],
    out_specs=[
        pl.BlockSpec((QROWS, PADK), lambda i: (i, 0)),
        pl.BlockSpec((QROWS, PADK), lambda i: (i, 0)),
    ],
    out_shape=[
        jax.ShapeDtypeStruct((VPACK, PADK), jnp.bfloat16),
        jax.ShapeDtypeStruct((VPACK, PADK), jnp.bfloat16),
    ],
)

_sc_mesh = plsc.VectorSubcoreMesh(core_axis_name="c", subcore_axis_name="s")


@functools.partial(
    pl.kernel,
    mesh=_sc_mesh,
    out_type=[
        jax.ShapeDtypeStruct((B, PADK), jnp.bfloat16),
        jax.ShapeDtypeStruct((B, PADK), jnp.bfloat16),
    ],
    scratch_types=[
        pltpu.VMEM((NCHUNK, CHUNK), jnp.int32),
        pltpu.VMEM((NCHUNK, CHUNK), jnp.int32),
        pltpu.VMEM((2, CHUNK, PADK), jnp.bfloat16),
        pltpu.VMEM((2, CHUNK, PADK), jnp.bfloat16),
        pltpu.SemaphoreType.DMA,
        pltpu.SemaphoreType.DMA,
    ],
    compiler_params=pltpu.CompilerParams(use_tc_tiling_on_sc=False),
)
def _sc_gather(uidx_hbm, iidx_hbm, utab_hbm, itab_hbm,
               uout_hbm, iout_hbm,
               uidx_v, iidx_v, urows_v, irows_v, sem_u, sem_i):
    wid = lax.axis_index("s") * NC + lax.axis_index("c")
    base = wid * BPW
    pltpu.sync_copy(uidx_hbm.at[wid], uidx_v)
    pltpu.sync_copy(iidx_hbm.at[wid], iidx_v)

    def fire(j):
        b = j % 2
        pltpu.async_copy(utab_hbm.at[uidx_v.at[j]], urows_v.at[b], sem_u)
        pltpu.async_copy(itab_hbm.at[iidx_v.at[j]], irows_v.at[b], sem_i)

    def drain_and_store(j):
        b = j % 2
        pltpu.make_async_copy(utab_hbm.at[uidx_v.at[j]],
                              urows_v.at[b], sem_u).wait()
        pltpu.make_async_copy(itab_hbm.at[iidx_v.at[j]],
                              irows_v.at[b], sem_i).wait()
        dst = pl.ds(base + j * CHUNK, CHUNK)
        pltpu.sync_copy(urows_v.at[b], uout_hbm.at[dst])
        pltpu.sync_copy(irows_v.at[b], iout_hbm.at[dst])

    fire(0)
    for j in range(NCHUNK):
        if j + 1 < NCHUNK:
            fire(j + 1)
        drain_and_store(j)


BLK = 2048
GRID = B // BLK


def _mlp_body(uep_ref, iep_ref, qu_ref, qi_ref,
              wcu_ref, wci_ref, bc_ref, wc2_ref,
              wtu_ref, wti_ref, bt_ref, wt2_ref,
              cvr_ref, ctr_ref):
    qu = qu_ref[...]
    qi = qi_ref[...]
    ue = jnp.zeros((BLK, K), jnp.float32)
    ie = jnp.zeros((BLK, K), jnp.float32)
    for q in range(4):
        ue = ue + jnp.where(
            qu == q, uep_ref[:, pl.ds(q * K, K)].astype(jnp.float32), 0.0)
        ie = ie + jnp.where(
            qi == q, iep_ref[:, pl.ds(q * K, K)].astype(jnp.float32), 0.0)
    h_cvr = jnp.maximum(
        jnp.dot(ue, wcu_ref[...], preferred_element_type=jnp.float32)
        + jnp.dot(ie, wci_ref[...], preferred_element_type=jnp.float32)
        + bc_ref[...], 0.0)
    cvr_ref[...] = jnp.dot(h_cvr, wc2_ref[...],
                           preferred_element_type=jnp.float32)
    h_ctr = jnp.maximum(
        jnp.dot(ue, wtu_ref[...], preferred_element_type=jnp.float32)
        + jnp.dot(ie, wti_ref[...], preferred_element_type=jnp.float32)
        + bt_ref[...], 0.0)
    ctr_ref[...] = jnp.dot(h_ctr, wt2_ref[...],
                           preferred_element_type=jnp.float32)


_full = lambda shape: pl.BlockSpec(shape, lambda i: (0, 0))
_rows = lambda shape: pl.BlockSpec(shape, lambda i: (i, 0))

_mlp_call = pl.pallas_call(
    _mlp_body,
    grid=(GRID,),
    in_specs=[
        _rows((BLK, PADK)), _rows((BLK, PADK)),
        _rows((BLK, 1)), _rows((BLK, 1)),
        _full((K, K)), _full((K, K)), _full((1, K)), _full((K, 1)),
        _full((K, K)), _full((K, K)), _full((1, K)), _full((K, 1)),
    ],
    out_specs=[_rows((BLK, 1)), _rows((BLK, 1))],
    out_shape=[
        jax.ShapeDtypeStruct((B, 1), jnp.float32),
        jax.ShapeDtypeStruct((B, 1), jnp.float32),
    ],
)


@jax.jit
def kernel(x, user_table, item_table, W_cvr1, b_cvr1, w_cvr2,
           W_ctr1, b_ctr1, w_ctr2):
    r_u = x[:, 0].astype(jnp.int32)
    r_i = x[:, 1].astype(jnp.int32)
    Ru = (r_u // TBLK) * QROWS + (r_u % QROWS)
    Ri = (r_i // TBLK) * QROWS + (r_i % QROWS)
    qu = ((r_u % TBLK) // QROWS).reshape(B, 1)
    qi = ((r_i % TBLK) // QROWS).reshape(B, 1)
    uidx = Ru.reshape(NW, NCHUNK, CHUNK)
    iidx = Ri.reshape(NW, NCHUNK, CHUNK)
    pmats = jnp.stack(
        [jnp.eye(K, PADK, k=q * K, dtype=jnp.bfloat16) for q in range(4)])
    utr, itr = _transpose_call(user_table.T, item_table.T, pmats)
    uep, iep = _sc_gather(uidx, iidx, utr, itr)
    cvr, ctr = _mlp_call(
        uep, iep, qu, qi,
        W_cvr1[:, :K].T, W_cvr1[:, K:].T, b_cvr1.reshape(1, K), w_cvr2.T,
        W_ctr1[:, :K].T, W_ctr1[:, K:].T, b_ctr1.reshape(1, K), w_ctr2.T,
    )
    return (cvr, ctr)


# speedup vs baseline: 1.8060x; 1.8060x over previous
import functools

import jax
import jax.numpy as jnp
from jax import lax
from jax.experimental import pallas as pl
from jax.experimental.pallas import tpu as pltpu
from jax.experimental.pallas import tpu_sc as plsc

B = 16384
K = 32
V = 1000000
PADK = 128
NC = 2
NS = 16
NW = NC * NS
BPW = B // NW
CHUNK = 128
NCHUNK = BPW // CHUNK

TBLK = 16384
QROWS = TBLK // 4
TGRID = (V + TBLK - 1) // TBLK
VPACK = TGRID * QROWS


def _transpose_body(ut_ref, it_ref, p_ref, uo_ref, io_ref):
    uo = io = None
    for q in range(4):
        pq = p_ref[q]
        du = lax.dot_general(
            ut_ref[:, pl.ds(q * QROWS, QROWS)], pq,
            (((0,), (0,)), ((), ())), preferred_element_type=jnp.float32)
        di = lax.dot_general(
            it_ref[:, pl.ds(q * QROWS, QROWS)], pq,
            (((0,), (0,)), ((), ())), preferred_element_type=jnp.float32)
        uo = du if uo is None else uo + du
        io = di if io is None else io + di
    uo_ref[...] = uo
    io_ref[...] = io


_transpose_call = pl.pallas_call(
    _transpose_body,
    grid=(TGRID,),
    in_specs=[
        pl.BlockSpec((K, TBLK), lambda i: (0, i)),
        pl.BlockSpec((K, TBLK), lambda i: (0, i)),
        pl.BlockSpec((4, K, PADK), lambda i: (0, 0, 0)),
    ],
    out_specs=[
        pl.BlockSpec((QROWS, PADK), lambda i: (i, 0)),
        pl.BlockSpec((QROWS, PADK), lambda i: (i, 0)),
    ],
    out_shape=[
        jax.ShapeDtypeStruct((VPACK, PADK), jnp.float32),
        jax.ShapeDtypeStruct((VPACK, PADK), jnp.float32),
    ],
    compiler_params=pltpu.CompilerParams(fuse_transposed_lhs_in_matmul=True),
)

_sc_mesh = plsc.VectorSubcoreMesh(core_axis_name="c", subcore_axis_name="s")


@functools.partial(
    pl.kernel,
    mesh=_sc_mesh,
    out_type=[
        jax.ShapeDtypeStruct((B, PADK), jnp.float32),
        jax.ShapeDtypeStruct((B, PADK), jnp.float32),
    ],
    scratch_types=[
        pltpu.VMEM((NCHUNK, CHUNK), jnp.int32),
        pltpu.VMEM((NCHUNK, CHUNK), jnp.int32),
        pltpu.VMEM((2, CHUNK, PADK), jnp.float32),
        pltpu.VMEM((2, CHUNK, PADK), jnp.float32),
        pltpu.SemaphoreType.DMA,
        pltpu.SemaphoreType.DMA,
    ],
    compiler_params=pltpu.CompilerParams(use_tc_tiling_on_sc=False),
)
def _sc_gather(uidx_hbm, iidx_hbm, utab_hbm, itab_hbm,
               uout_hbm, iout_hbm,
               uidx_v, iidx_v, urows_v, irows_v, sem_u, sem_i):
    wid = lax.axis_index("s") * NC + lax.axis_index("c")
    base = wid * BPW
    pltpu.sync_copy(uidx_hbm.at[wid], uidx_v)
    pltpu.sync_copy(iidx_hbm.at[wid], iidx_v)

    def fire(j):
        b = j % 2
        pltpu.async_copy(utab_hbm.at[uidx_v.at[j]], urows_v.at[b], sem_u)
        pltpu.async_copy(itab_hbm.at[iidx_v.at[j]], irows_v.at[b], sem_i)

    def drain_and_store(j):
        b = j % 2
        pltpu.make_async_copy(utab_hbm.at[uidx_v.at[j]],
                              urows_v.at[b], sem_u).wait()
        pltpu.make_async_copy(itab_hbm.at[iidx_v.at[j]],
                              irows_v.at[b], sem_i).wait()
        dst = pl.ds(base + j * CHUNK, CHUNK)
        pltpu.sync_copy(urows_v.at[b], uout_hbm.at[dst])
        pltpu.sync_copy(irows_v.at[b], iout_hbm.at[dst])

    fire(0)
    for j in range(NCHUNK):
        if j + 1 < NCHUNK:
            fire(j + 1)
        drain_and_store(j)


BLK = 2048
GRID = B // BLK


def _mlp_body(uep_ref, iep_ref, qu_ref, qi_ref,
              wcu_ref, wci_ref, bc_ref, wc2_ref,
              wtu_ref, wti_ref, bt_ref, wt2_ref,
              cvr_ref, ctr_ref):
    qu = qu_ref[...]
    qi = qi_ref[...]
    ue = jnp.zeros((BLK, K), jnp.float32)
    ie = jnp.zeros((BLK, K), jnp.float32)
    for q in range(4):
        ue = ue + jnp.where(qu == q, uep_ref[:, pl.ds(q * K, K)], 0.0)
        ie = ie + jnp.where(qi == q, iep_ref[:, pl.ds(q * K, K)], 0.0)
    h_cvr = jnp.maximum(
        jnp.dot(ue, wcu_ref[...], preferred_element_type=jnp.float32)
        + jnp.dot(ie, wci_ref[...], preferred_element_type=jnp.float32)
        + bc_ref[...], 0.0)
    cvr_ref[...] = jnp.dot(h_cvr, wc2_ref[...],
                           preferred_element_type=jnp.float32)
    h_ctr = jnp.maximum(
        jnp.dot(ue, wtu_ref[...], preferred_element_type=jnp.float32)
        + jnp.dot(ie, wti_ref[...], preferred_element_type=jnp.float32)
        + bt_ref[...], 0.0)
    ctr_ref[...] = jnp.dot(h_ctr, wt2_ref[...],
                           preferred_element_type=jnp.float32)


_full = lambda shape: pl.BlockSpec(shape, lambda i: (0, 0))
_rows = lambda shape: pl.BlockSpec(shape, lambda i: (i, 0))

_mlp_call = pl.pallas_call(
    _mlp_body,
    grid=(GRID,),
    in_specs=[
        _rows((BLK, PADK)), _rows((BLK, PADK)),
        _rows((BLK, 1)), _rows((BLK, 1)),
        _full((K, K)), _full((K, K)), _full((1, K)), _full((K, 1)),
        _full((K, K)), _full((K, K)), _full((1, K)), _full((K, 1)),
    ],
    out_specs=[_rows((BLK, 1)), _rows((BLK, 1))],
    out_shape=[
        jax.ShapeDtypeStruct((B, 1), jnp.float32),
        jax.ShapeDtypeStruct((B, 1), jnp.float32),
    ],
)


@jax.jit
def kernel(x, user_table, item_table, W_cvr1, b_cvr1, w_cvr2,
           W_ctr1, b_ctr1, w_ctr2):
    r_u = x[:, 0].astype(jnp.int32)
    r_i = x[:, 1].astype(jnp.int32)
    Ru = (r_u // TBLK) * QROWS + (r_u % QROWS)
    Ri = (r_i // TBLK) * QROWS + (r_i % QROWS)
    qu = ((r_u % TBLK) // QROWS).reshape(B, 1)
    qi = ((r_i % TBLK) // QROWS).reshape(B, 1)
    uidx = Ru.reshape(NW, NCHUNK, CHUNK)
    iidx = Ri.reshape(NW, NCHUNK, CHUNK)
    pmats = jnp.stack(
        [jnp.eye(K, PADK, k=q * K, dtype=jnp.float32) for q in range(4)])
    utr, itr = _transpose_call(user_table.T, item_table.T, pmats)
    uep, iep = _sc_gather(uidx, iidx, utr, itr)
    cvr, ctr = _mlp_call(
        uep, iep, qu, qi,
        W_cvr1[:, :K].T, W_cvr1[:, K:].T, b_cvr1.reshape(1, K), w_cvr2.T,
        W_ctr1[:, :K].T, W_ctr1[:, K:].T, b_ctr1.reshape(1, K), w_ctr2.T,
    )
    return (cvr, ctr)
